# Initial kernel scaffold; baseline (speedup 1.0000x reference)
import jax
import jax.numpy as jnp
from jax.experimental import pallas as pl


def kernel(sentence_embedding, W, b_pre, latent_bias):
    raise NotImplementedError("write your pallas kernel here")


# speedup vs baseline: 10.0604x; 10.0604x over previous
import functools

import jax
import jax.numpy as jnp
from jax.experimental import pallas as pl
from jax.experimental.pallas import tpu as pltpu


def _encode_body(x_ref, w_ref, bpre_ref, lb_ref, z_ref):
    x = x_ref[...] - bpre_ref[...]
    z = jax.lax.dot_general(
        x, w_ref[...], (((1,), (1,)), ((), ())),
        preferred_element_type=jnp.float32)
    z_ref[...] = z + lb_ref[...]


def _encode(x, W, b_pre, latent_bias, bm, bn):
    B, D = x.shape
    H = W.shape[0]
    return pl.pallas_call(
        _encode_body,
        grid=(B // bm, H // bn),
        in_specs=[
            pl.BlockSpec((bm, D), lambda i, j: (i, 0)),
            pl.BlockSpec((bn, D), lambda i, j: (j, 0)),
            pl.BlockSpec((1, D), lambda i, j: (0, 0)),
            pl.BlockSpec((1, bn), lambda i, j: (0, j)),
        ],
        out_specs=pl.BlockSpec((bm, bn), lambda i, j: (i, j)),
        out_shape=jax.ShapeDtypeStruct((B, H), jnp.float32),
        compiler_params=pltpu.CompilerParams(
            dimension_semantics=("parallel", "parallel")),
    )(x, W, b_pre.reshape(1, D), latent_bias.reshape(1, H))


def _select_body(z_ref, zk_ref, z4k_ref, zaux_ref, *, ks):
    z = z_ref[...]
    raw = jax.lax.bitcast_convert_type(z, jnp.int32)
    keys = jnp.where(raw < 0, jnp.bitwise_xor(raw, jnp.int32(0x7FFFFFFF)), raw)

    k4, k1, k0 = ks

    def count_ge(t):
        return jnp.sum((keys >= t).astype(jnp.int32), axis=1, keepdims=True)

    rows = z.shape[0]
    nneg = count_ge(jnp.zeros((rows, 1), dtype=jnp.int32))
    imin = jnp.int32(jnp.iinfo(jnp.int32).min)
    zero32 = jnp.int32(0)

    def sign_init(k):
        return jnp.where(nneg >= k, zero32, imin)

    def body(i, carry):
        t4, t1, t0 = carry
        bit = jnp.right_shift(jnp.int32(1 << 30), i)
        try4 = t4 + bit
        try1 = t1 + bit
        try0 = t0 + bit
        t4 = jnp.where(count_ge(try4) >= k4, try4, t4)
        t1 = jnp.where(count_ge(try1) >= k1, try1, t1)
        t0 = jnp.where(count_ge(try0) >= k0, try0, t0)
        return (t4, t1, t0)

    t4, t1, t0 = jax.lax.fori_loop(
        0, 31, body, (sign_init(k4), sign_init(k1), sign_init(k0)))

    zero = jnp.zeros_like(z)
    z4k_ref[...] = jnp.where(keys >= t4, z, zero)
    zk_ref[...] = jnp.where(keys >= t1, z, zero)
    zaux_ref[...] = jnp.where(keys >= t0, z, zero)


def _select(z, ks, bm):
    B, H = z.shape
    out = jax.ShapeDtypeStruct((B, H), jnp.float32)
    return pl.pallas_call(
        functools.partial(_select_body, ks=ks),
        grid=(B // bm,),
        in_specs=[pl.BlockSpec((bm, H), lambda i: (i, 0))],
        out_specs=[pl.BlockSpec((bm, H), lambda i: (i, 0))] * 3,
        out_shape=[out, out, out],
        compiler_params=pltpu.CompilerParams(
            dimension_semantics=("parallel",)),
    )(z)


def _decode_body(zk_ref, z4k_ref, zaux_ref, w_ref, x_ref, bpre_ref,
                 xk_ref, x4k_ref, xaux_ref, e_ref, ehat_ref, *, nj):
    j = pl.program_id(1)

    def mm(a_ref):
        return jax.lax.dot_general(
            a_ref[...], w_ref[...], (((1,), (0,)), ((), ())),
            preferred_element_type=jnp.float32)

    @pl.when(j == 0)
    def _init():
        xk_ref[...] = mm(zk_ref)
        x4k_ref[...] = mm(z4k_ref)
        xaux_ref[...] = mm(zaux_ref)

    @pl.when(j > 0)
    def _acc():
        xk_ref[...] += mm(zk_ref)
        x4k_ref[...] += mm(z4k_ref)
        xaux_ref[...] += mm(zaux_ref)

    @pl.when(j == nj - 1)
    def _fin():
        b = bpre_ref[...]
        xk = xk_ref[...] + b
        x4k = x4k_ref[...] + b
        xaux = xaux_ref[...] + b
        xk_ref[...] = xk
        x4k_ref[...] = x4k
        xaux_ref[...] = xaux
        e_ref[...] = x_ref[...] - xaux
        ehat_ref[...] = xk + b


def _decode(zk, z4k, zaux, W, x, b_pre, bm, bk):
    B, H = zk.shape
    D = W.shape[1]
    nj = H // bk
    out = jax.ShapeDtypeStruct((B, D), jnp.float32)
    zspec = pl.BlockSpec((bm, bk), lambda i, j: (i, j))
    return pl.pallas_call(
        functools.partial(_decode_body, nj=nj),
        grid=(B // bm, nj),
        in_specs=[
            zspec, zspec, zspec,
            pl.BlockSpec((bk, D), lambda i, j: (j, 0)),
            pl.BlockSpec((bm, D), lambda i, j: (i, 0)),
            pl.BlockSpec((1, D), lambda i, j: (0, 0)),
        ],
        out_specs=[pl.BlockSpec((bm, D), lambda i, j: (i, 0))] * 5,
        out_shape=[out] * 5,
        compiler_params=pltpu.CompilerParams(
            dimension_semantics=("parallel", "arbitrary")),
    )(zk, z4k, zaux, W, x, b_pre.reshape(1, D))


def kernel(sentence_embedding, W, b_pre, latent_bias):
    x = sentence_embedding
    B, D = x.shape
    H = W.shape[0]

    z = _encode(x, W, b_pre, latent_bias, bm=256, bn=2048)
    zk, z4k, zaux = _select(z, ks=(400, 100, 50), bm=128)
    xk, x4k, xaux, e, ehat = _decode(zk, z4k, zaux, W, x, b_pre,
                                     bm=256, bk=2048)
    return (zk, x, z, z4k, zaux, xk, x4k, xaux, e, ehat)
